# Initial kernel scaffold; baseline (speedup 1.0000x reference)
import jax
import jax.numpy as jnp
from jax.experimental import pallas as pl


def kernel(weather_ids, time_of_day, season_ids, W_weather, W_season, W_t1, b_t1, W_t2, b_t2, W_c, b_c):
    raise NotImplementedError("write your pallas kernel here")


# speedup vs baseline: 2.2987x; 2.2987x over previous
import functools

import jax
import jax.numpy as jnp
from jax import lax
from jax.experimental import pallas as pl
from jax.experimental.pallas import tpu as pltpu
from jax.experimental.pallas import tpu_sc as plsc

_B, _L = 16384, 50
_WEATHER, _SEASON, _TDIM, _EDIM = 10, 4, 8, 64
_P = _B * _L
_NSEG = _TDIM + 1
_NJ = _WEATHER * _SEASON * _NSEG

_NC, _NS = 2, 16
_NW = _NC * _NS
_PPW = _P // _NW
_CH = 1024
_NCHUNK = _PPW // _CH


def _sc_body(wid_hbm, sid_hbm, t_hbm, j_hbm, a_hbm, th_hbm, out_hbm,
             wid_v, sid_v, t_v, outb, j_v, a_v, th_v):
    w = lax.axis_index("s") * _NC + lax.axis_index("c")
    pltpu.sync_copy(j_hbm, j_v)
    pltpu.sync_copy(a_hbm, a_v)
    pltpu.sync_copy(th_hbm, th_v)
    lane64 = lax.iota(jnp.int32, 16) * _EDIM
    th_vecs = [th_v[pl.ds(i * 16, 16)] for i in range(_TDIM)]
    base0 = w * _PPW

    def chunk_body(ci, carry):
        pos0 = base0 + ci * _CH
        pltpu.sync_copy(wid_hbm.at[pl.ds(pos0, _CH)], wid_v)
        pltpu.sync_copy(sid_hbm.at[pl.ds(pos0, _CH)], sid_v)
        pltpu.sync_copy(t_hbm.at[pl.ds(pos0, _CH)], t_v)

        def group_body(g, c2):
            gb = g * 16
            wv = wid_v[pl.ds(gb, 16)]
            sv = sid_v[pl.ds(gb, 16)]
            tv = t_v[pl.ds(gb, 16)]
            seg = jnp.where(tv > th_vecs[0], 1, 0)
            for i in range(1, _TDIM):
                seg = seg + jnp.where(tv > th_vecs[i], 1, 0)
            jb = wv * (_SEASON * _NSEG * _EDIM) + sv * (_NSEG * _EDIM) + seg * _EDIM
            ab = seg * _EDIM
            ob = lane64 + gb * _EDIM
            for d in range(_EDIM):
                jv = plsc.load_gather(j_v, [jb + d])
                av = plsc.load_gather(a_v, [ab + d])
                plsc.store_scatter(outb, [ob + d], jv + tv * av)
            return c2

        lax.fori_loop(0, _CH // 16, group_body, 0)
        pltpu.sync_copy(outb, out_hbm.at[pl.ds(pos0 * _EDIM, _CH * _EDIM)])
        return carry

    lax.fori_loop(0, _NCHUNK, chunk_body, 0)


_mesh = plsc.VectorSubcoreMesh(core_axis_name="c", subcore_axis_name="s")
_sc_call = pl.kernel(
    _sc_body,
    out_type=jax.ShapeDtypeStruct((_P * _EDIM,), jnp.float32),
    mesh=_mesh,
    compiler_params=pltpu.CompilerParams(needs_layout_passes=False),
    scratch_types=[
        pltpu.VMEM((_CH,), jnp.int32),
        pltpu.VMEM((_CH,), jnp.int32),
        pltpu.VMEM((_CH,), jnp.float32),
        pltpu.VMEM((_CH * _EDIM,), jnp.float32),
        pltpu.VMEM((_NJ * _EDIM,), jnp.float32),
        pltpu.VMEM((_NSEG * _EDIM,), jnp.float32),
        pltpu.VMEM((_TDIM * 16,), jnp.float32),
    ],
)


def _fold_tables(W_weather, W_season, W_t1, b_t1, W_t2, b_t2, W_c, b_c):
    f32 = jnp.float32
    Wc = W_c.astype(f32)
    Tw = W_weather.astype(f32) @ Wc[0:16]
    Ts = W_season.astype(f32) @ Wc[16:32]
    W2c = W_t2.astype(f32) @ Wc[32:64]
    btot = b_t2.astype(f32) @ Wc[32:64] + b_c.astype(f32)
    w1 = W_t1.astype(f32)[0]
    b1 = b_t1.astype(f32)
    safe_w1 = jnp.where(w1 != 0, w1, 1.0)
    theta = jnp.where(w1 != 0, -b1 / safe_w1, jnp.inf)
    order = jnp.argsort(theta)
    theta_s = theta[order]
    rank = jnp.argsort(order)
    k = jnp.arange(_NSEG)[:, None]
    active = jnp.where(w1[None, :] > 0, rank[None, :] < k,
                       jnp.where(w1[None, :] < 0, rank[None, :] >= k,
                                 b1[None, :] > 0))
    act = active.astype(f32)
    A = (act * w1[None, :]) @ W2c
    Bs = (act * b1[None, :]) @ W2c + btot
    Tws = (Tw[:, None, :] + Ts[None, :, :]).reshape(_WEATHER * _SEASON, _EDIM)
    J = (Tws[:, None, :] + Bs[None, :, :]).reshape(_NJ, _EDIM)
    return J, A, theta_s


def kernel(weather_ids, time_of_day, season_ids, W_weather, W_season,
           W_t1, b_t1, W_t2, b_t2, W_c, b_c):
    J, A, theta_s = _fold_tables(W_weather, W_season, W_t1, b_t1,
                                 W_t2, b_t2, W_c, b_c)
    wid = weather_ids.reshape(_P).astype(jnp.int32)
    sid = season_ids.reshape(_P).astype(jnp.int32)
    t = time_of_day.reshape(_P).astype(jnp.float32)
    jf = J.reshape(_NJ * _EDIM)
    af = A.reshape(_NSEG * _EDIM)
    thb = jnp.broadcast_to(theta_s[:, None], (_TDIM, 16)).reshape(_TDIM * 16)
    out = _sc_call(wid, sid, t, jf, af, thb)
    return out.reshape(_B, _L, _EDIM)


# speedup vs baseline: 3.1852x; 1.3857x over previous
import functools

import jax
import jax.numpy as jnp
from jax import lax
from jax.experimental import pallas as pl
from jax.experimental.pallas import tpu as pltpu
from jax.experimental.pallas import tpu_sc as plsc

_B, _L = 16384, 50
_WEATHER, _SEASON, _TDIM, _EDIM = 10, 4, 8, 64
_P = _B * _L
_NSEG = _TDIM + 1
_NJ = _WEATHER * _SEASON * _NSEG

_NC, _NS = 2, 16
_NW = _NC * _NS
_PPW = _P // _NW
_CH = 512
_NCHUNK = _PPW // _CH


def _sc_body(wid_hbm, sid_hbm, t_hbm, j_hbm, a_hbm, th_hbm, out_hbm,
             w0, w1, s0, s1, t0, t1, o0, o1, j_v, a_v, th_v,
             si0, si1, so0, so1):
    wkr = lax.axis_index("s") * _NC + lax.axis_index("c")
    base0 = wkr * _PPW
    ins = ((w0, s0, t0, si0), (w1, s1, t1, si1))
    outs = ((o0, so0), (o1, so1))

    def start_in(ci, b):
        pos0 = base0 + ci * _CH
        pltpu.async_copy(wid_hbm.at[pl.ds(pos0, _CH)], ins[b][0], ins[b][3])
        pltpu.async_copy(sid_hbm.at[pl.ds(pos0, _CH)], ins[b][1], ins[b][3])
        pltpu.async_copy(t_hbm.at[pl.ds(pos0, _CH)], ins[b][2], ins[b][3])

    def wait_in(b):
        for r in ins[b][0:3]:
            pltpu.make_async_copy(wid_hbm.at[pl.ds(base0, _CH)], r,
                                  ins[b][3]).wait()

    def wait_out(b):
        pltpu.make_async_copy(outs[b][0],
                              out_hbm.at[pl.ds(base0 * _EDIM, _CH * _EDIM)],
                              outs[b][1]).wait()

    start_in(0, 0)
    start_in(1, 1)
    pltpu.sync_copy(j_hbm, j_v)
    pltpu.sync_copy(a_hbm, a_v)
    pltpu.sync_copy(th_hbm, th_v)

    lane64 = lax.iota(jnp.int32, 16) * _EDIM
    th_vecs = [th_v[pl.ds(i * 16, 16)] for i in range(_TDIM)]

    def compute(b):
        widb, sidb, tb, _ = ins[b]
        outb = outs[b][0]

        @plsc.parallel_loop(0, _CH // 16, unroll=2)
        def group(g):
            gb = g * 16
            wv = widb[pl.ds(gb, 16)]
            sv = sidb[pl.ds(gb, 16)]
            tv = tb[pl.ds(gb, 16)]
            seg = jnp.where(tv > th_vecs[0], 1, 0)
            for i in range(1, _TDIM):
                seg = seg + jnp.where(tv > th_vecs[i], 1, 0)
            jb = (wv * (_SEASON * _NSEG * _EDIM) + sv * (_NSEG * _EDIM)
                  + seg * _EDIM)
            ab = seg * _EDIM
            ob = lane64 + gb * _EDIM
            for d0 in range(0, _EDIM, 8):
                js = [plsc.load_gather(j_v, [jb + (d0 + k)]) for k in range(8)]
                avs = [plsc.load_gather(a_v, [ab + (d0 + k)]) for k in range(8)]
                vals = [js[k] + tv * avs[k] for k in range(8)]
                for k in range(8):
                    plsc.store_scatter(outb, [ob + (d0 + k)], vals[k])

    def outer(oi, carry):
        for b in range(2):
            ci = oi * 2 + b
            pos0 = base0 + ci * _CH

            @pl.when(ci >= 2)
            def _():
                wait_out(b)

            wait_in(b)
            compute(b)
            pltpu.async_copy(outs[b][0],
                             out_hbm.at[pl.ds(pos0 * _EDIM, _CH * _EDIM)],
                             outs[b][1])

            @pl.when(ci + 2 < _NCHUNK)
            def _():
                start_in(ci + 2, b)
        return carry

    lax.fori_loop(0, _NCHUNK // 2, outer, 0)
    wait_out(0)
    wait_out(1)


_mesh = plsc.VectorSubcoreMesh(core_axis_name="c", subcore_axis_name="s")
_sc_call = pl.kernel(
    _sc_body,
    out_type=jax.ShapeDtypeStruct((_P * _EDIM,), jnp.float32),
    mesh=_mesh,
    compiler_params=pltpu.CompilerParams(needs_layout_passes=False),
    scratch_types=[
        pltpu.VMEM((_CH,), jnp.int32), pltpu.VMEM((_CH,), jnp.int32),
        pltpu.VMEM((_CH,), jnp.int32), pltpu.VMEM((_CH,), jnp.int32),
        pltpu.VMEM((_CH,), jnp.float32), pltpu.VMEM((_CH,), jnp.float32),
        pltpu.VMEM((_CH * _EDIM,), jnp.float32),
        pltpu.VMEM((_CH * _EDIM,), jnp.float32),
        pltpu.VMEM((_NJ * _EDIM,), jnp.float32),
        pltpu.VMEM((_NSEG * _EDIM,), jnp.float32),
        pltpu.VMEM((_TDIM * 16,), jnp.float32),
        pltpu.SemaphoreType.DMA, pltpu.SemaphoreType.DMA,
        pltpu.SemaphoreType.DMA, pltpu.SemaphoreType.DMA,
    ],
)


def _fold_tables(W_weather, W_season, W_t1, b_t1, W_t2, b_t2, W_c, b_c):
    f32 = jnp.float32
    Wc = W_c.astype(f32)
    Tw = W_weather.astype(f32) @ Wc[0:16]
    Ts = W_season.astype(f32) @ Wc[16:32]
    W2c = W_t2.astype(f32) @ Wc[32:64]
    btot = b_t2.astype(f32) @ Wc[32:64] + b_c.astype(f32)
    w1 = W_t1.astype(f32)[0]
    b1 = b_t1.astype(f32)
    safe_w1 = jnp.where(w1 != 0, w1, 1.0)
    theta = jnp.where(w1 != 0, -b1 / safe_w1, jnp.inf)
    order = jnp.argsort(theta)
    theta_s = theta[order]
    rank = jnp.argsort(order)
    k = jnp.arange(_NSEG)[:, None]
    active = jnp.where(w1[None, :] > 0, rank[None, :] < k,
                       jnp.where(w1[None, :] < 0, rank[None, :] >= k,
                                 b1[None, :] > 0))
    act = active.astype(f32)
    A = (act * w1[None, :]) @ W2c
    Bs = (act * b1[None, :]) @ W2c + btot
    Tws = (Tw[:, None, :] + Ts[None, :, :]).reshape(_WEATHER * _SEASON, _EDIM)
    J = (Tws[:, None, :] + Bs[None, :, :]).reshape(_NJ, _EDIM)
    return J, A, theta_s


def kernel(weather_ids, time_of_day, season_ids, W_weather, W_season,
           W_t1, b_t1, W_t2, b_t2, W_c, b_c):
    J, A, theta_s = _fold_tables(W_weather, W_season, W_t1, b_t1,
                                 W_t2, b_t2, W_c, b_c)
    wid = weather_ids.reshape(_P).astype(jnp.int32)
    sid = season_ids.reshape(_P).astype(jnp.int32)
    t = time_of_day.reshape(_P).astype(jnp.float32)
    jf = J.reshape(_NJ * _EDIM)
    af = A.reshape(_NSEG * _EDIM)
    thb = jnp.broadcast_to(theta_s[:, None], (_TDIM, 16)).reshape(_TDIM * 16)
    out = _sc_call(wid, sid, t, jf, af, thb)
    return out.reshape(_B, _L, _EDIM)
